# baseline (speedup 1.0000x reference)
import functools

import jax
import jax.numpy as jnp
from jax import lax
from jax.experimental import pallas as pl
from jax.experimental.pallas import tpu as pltpu
from jax.experimental.pallas import tpu_sc as plsc

_D_IN = 8192
_D_F = 1024
_B = 4096
_NC = 2
_NS = 16
_NW = _NC * _NS
_RPW = _B // _NW
_L = 16
_GROUPS = _D_IN // _L
_RB = 4
_NBLK = _RPW // _RB

_mesh = plsc.VectorSubcoreMesh(core_axis_name="c", subcore_axis_name="s")


@functools.partial(
    pl.kernel,
    out_type=jax.ShapeDtypeStruct((_B, _D_F), jnp.float32),
    mesh=_mesh,
    compiler_params=pltpu.CompilerParams(needs_layout_passes=False),
    scratch_types=(
        [pltpu.VMEM((_D_IN,), jnp.int32),
         pltpu.VMEM((_D_IN,), jnp.float32)]
        + [pltpu.VMEM((_D_IN,), jnp.float32) for _ in range(2 * _RB)]
        + [pltpu.VMEM((_D_F,), jnp.float32) for _ in range(2 * _RB)]
        + [pltpu.SemaphoreType.DMA for _ in range(4)]
    ),
)
def _countsketch(x_hbm, s_hbm, i_hbm, out_hbm, idx_v, sgn_v, *bufs):
    x_refs = (bufs[0:_RB], bufs[_RB:2 * _RB])
    acc_refs = (bufs[2 * _RB:3 * _RB], bufs[3 * _RB:4 * _RB])
    sem_x = bufs[4 * _RB:4 * _RB + 2]
    sem_o = bufs[4 * _RB + 2:4 * _RB + 4]
    wid = lax.axis_index("s") * _NC + lax.axis_index("c")
    base = wid * _RPW
    pltpu.sync_copy(i_hbm, idx_v)
    pltpu.sync_copy(s_hbm, sgn_v)

    def start_x(bi, s):
        row0 = base + bi * _RB
        for r in range(_RB):
            pltpu.async_copy(x_hbm.at[row0 + r], x_refs[s][r], sem_x[s])

    def wait_x(s):
        for r in range(_RB):
            pltpu.make_async_copy(x_hbm.at[base], x_refs[s][r], sem_x[s]).wait()

    def start_out(bi, s):
        row0 = base + bi * _RB
        for r in range(_RB):
            pltpu.async_copy(acc_refs[s][r], out_hbm.at[row0 + r], sem_o[s])

    def wait_out(s):
        for r in range(_RB):
            pltpu.make_async_copy(acc_refs[s][r], out_hbm.at[base], sem_o[s]).wait()

    start_x(0, 0)

    def pair_body(p, carry):
        for par in (0, 1):
            bi = p * 2 + par
            nbi = jnp.minimum(bi + 1, _NBLK - 1)
            start_x(nbi, 1 - par)
            wait_x(par)

            @pl.when(bi >= 2)
            def _():
                wait_out(par)

            @plsc.parallel_loop(0, _D_F // _L, unroll=4)
            def zero_body(k):
                z = jnp.zeros((_L,), jnp.float32)
                for r in range(_RB):
                    acc_refs[par][r][pl.ds(k * _L, _L)] = z

            @plsc.parallel_loop(0, _GROUPS, unroll=8)
            def scat_body(j):
                o = j * _L
                idx = idx_v[pl.ds(o, _L)]
                sgn = sgn_v[pl.ds(o, _L)]
                for r in range(_RB):
                    v = x_refs[par][r][pl.ds(o, _L)] * sgn
                    plsc.addupdate_scatter(acc_refs[par][r], [idx], v)
            start_out(bi, par)
        return carry

    lax.fori_loop(0, _NBLK // 2, pair_body, 0)
    wait_x(0)
    wait_out(0)
    wait_out(1)


def kernel(x, s_hash, i_hash):
    return _countsketch(x, s_hash, i_hash.astype(jnp.int32))


# speedup vs baseline: 1.2819x; 1.2819x over previous
import functools

import jax
import jax.numpy as jnp
from jax import lax
from jax.experimental import pallas as pl
from jax.experimental.pallas import tpu as pltpu
from jax.experimental.pallas import tpu_sc as plsc

_D_IN = 8192
_D_F = 1024
_B = 4096
_B_SC = 2560
_B_TC = _B - _B_SC

_NC = 2
_NS = 16
_NW = _NC * _NS
_RPW = _B_SC // _NW
_L = 16
_GROUPS = _D_IN // _L
_RB = 4
_NBLK = _RPW // _RB

_mesh = plsc.VectorSubcoreMesh(core_axis_name="c", subcore_axis_name="s")


@functools.partial(
    pl.kernel,
    out_type=jax.ShapeDtypeStruct((_B_SC, _D_F), jnp.float32),
    mesh=_mesh,
    compiler_params=pltpu.CompilerParams(needs_layout_passes=False),
    scratch_types=(
        [pltpu.VMEM((_D_IN,), jnp.int32),
         pltpu.VMEM((_D_IN,), jnp.float32)]
        + [pltpu.VMEM((_D_IN,), jnp.float32) for _ in range(2 * _RB)]
        + [pltpu.VMEM((_D_F,), jnp.float32) for _ in range(2 * _RB)]
        + [pltpu.SemaphoreType.DMA for _ in range(4)]
    ),
)
def _sc_sketch(x_hbm, s_hbm, i_hbm, out_hbm, idx_v, sgn_v, *bufs):
    x_refs = (bufs[0:_RB], bufs[_RB:2 * _RB])
    acc_refs = (bufs[2 * _RB:3 * _RB], bufs[3 * _RB:4 * _RB])
    sem_x = bufs[4 * _RB:4 * _RB + 2]
    sem_o = bufs[4 * _RB + 2:4 * _RB + 4]
    wid = lax.axis_index("s") * _NC + lax.axis_index("c")
    base = wid * _RPW
    pltpu.sync_copy(i_hbm, idx_v)
    pltpu.sync_copy(s_hbm, sgn_v)

    def start_x(bi, s):
        row0 = base + bi * _RB
        for r in range(_RB):
            pltpu.async_copy(x_hbm.at[row0 + r], x_refs[s][r], sem_x[s])

    def wait_x(s):
        for r in range(_RB):
            pltpu.make_async_copy(x_hbm.at[base], x_refs[s][r], sem_x[s]).wait()

    def start_out(bi, s):
        row0 = base + bi * _RB
        for r in range(_RB):
            pltpu.async_copy(acc_refs[s][r], out_hbm.at[row0 + r], sem_o[s])

    def wait_out(s):
        for r in range(_RB):
            pltpu.make_async_copy(acc_refs[s][r], out_hbm.at[base], sem_o[s]).wait()

    start_x(0, 0)

    def pair_body(p, carry):
        for par in (0, 1):
            bi = p * 2 + par
            nbi = jnp.minimum(bi + 1, _NBLK - 1)
            start_x(nbi, 1 - par)

            @pl.when(bi >= 2)
            def _():
                wait_out(par)

            @plsc.parallel_loop(0, _D_F // _L, unroll=4)
            def zero_body(k):
                z = jnp.zeros((_L,), jnp.float32)
                for r in range(_RB):
                    acc_refs[par][r][pl.ds(k * _L, _L)] = z

            wait_x(par)

            @plsc.parallel_loop(0, _GROUPS, unroll=4)
            def scat_body(j):
                o = j * _L
                idx = idx_v[pl.ds(o, _L)]
                sgn = sgn_v[pl.ds(o, _L)]
                for r in range(_RB):
                    v = x_refs[par][r][pl.ds(o, _L)] * sgn
                    plsc.addupdate_scatter(acc_refs[par][r], [idx], v)
            start_out(bi, par)
        return carry

    lax.fori_loop(0, _NBLK // 2, pair_body, 0)
    wait_x(0)
    wait_out(0)
    wait_out(1)


_BM = 256
_BK = 512
_NKB = _D_IN // _BK
_ROW0_BLK = _B_SC // _BM


def _tc_body(i2d_ref, s2d_ref, x_ref, o_ref):
    k = pl.program_id(1)

    @pl.when(k == 0)
    def _():
        o_ref[...] = jnp.zeros_like(o_ref)

    ih = i2d_ref[pl.ds(k * _BK, _BK), :]
    sg = s2d_ref[pl.ds(k * _BK, _BK), :]
    iota = lax.broadcasted_iota(jnp.int32, (_BK, _D_F), 1)
    s_blk = jnp.where(iota == ih, sg, 0.0)
    o_ref[...] += jnp.dot(x_ref[...], s_blk,
                          preferred_element_type=jnp.float32)


def _tc_sketch(x, s_hash, i_hash):
    return pl.pallas_call(
        _tc_body,
        grid=(_B_TC // _BM, _NKB),
        in_specs=[
            pl.BlockSpec((_D_IN, 1), lambda i, k: (0, 0)),
            pl.BlockSpec((_D_IN, 1), lambda i, k: (0, 0)),
            pl.BlockSpec((_BM, _BK), lambda i, k: (i + _ROW0_BLK, k)),
        ],
        out_specs=pl.BlockSpec((_BM, _D_F), lambda i, k: (i, 0)),
        out_shape=jax.ShapeDtypeStruct((_B_TC, _D_F), jnp.float32),
    )(i_hash.reshape(_D_IN, 1), s_hash.reshape(_D_IN, 1), x)


def kernel(x, s_hash, i_hash):
    i32 = i_hash.astype(jnp.int32)
    out_sc = _sc_sketch(x, s_hash, i32)
    out_tc = _tc_sketch(x, s_hash, i32)
    return jnp.concatenate([out_sc, out_tc], axis=0)


# speedup vs baseline: 1.4808x; 1.1552x over previous
import functools

import jax
import jax.numpy as jnp
from jax import lax
from jax.experimental import pallas as pl
from jax.experimental.pallas import tpu as pltpu
from jax.experimental.pallas import tpu_sc as plsc

_D_IN = 8192
_D_F = 1024
_B = 4096
_B_SC = 2048
_B_TC = _B - _B_SC

_NC = 2
_NS = 16
_NW = _NC * _NS
_RPW = _B_SC // _NW
_L = 16
_GROUPS = _D_IN // _L
_RB = 4
_NBLK = _RPW // _RB

_mesh = plsc.VectorSubcoreMesh(core_axis_name="c", subcore_axis_name="s")


@functools.partial(
    pl.kernel,
    out_type=jax.ShapeDtypeStruct((_B_SC, _D_F), jnp.float32),
    mesh=_mesh,
    compiler_params=pltpu.CompilerParams(needs_layout_passes=False),
    scratch_types=(
        [pltpu.VMEM((_D_IN,), jnp.int32),
         pltpu.VMEM((_D_IN,), jnp.float32)]
        + [pltpu.VMEM((_D_IN,), jnp.float32) for _ in range(2 * _RB)]
        + [pltpu.VMEM((_D_F,), jnp.float32) for _ in range(2 * _RB)]
        + [pltpu.SemaphoreType.DMA for _ in range(4)]
    ),
)
def _sc_sketch(x_hbm, s_hbm, i_hbm, out_hbm, idx_v, sgn_v, *bufs):
    x_refs = (bufs[0:_RB], bufs[_RB:2 * _RB])
    acc_refs = (bufs[2 * _RB:3 * _RB], bufs[3 * _RB:4 * _RB])
    sem_x = bufs[4 * _RB:4 * _RB + 2]
    sem_o = bufs[4 * _RB + 2:4 * _RB + 4]
    wid = lax.axis_index("s") * _NC + lax.axis_index("c")
    base = wid * _RPW
    pltpu.sync_copy(i_hbm, idx_v)
    pltpu.sync_copy(s_hbm, sgn_v)

    def start_x(bi, s):
        row0 = base + bi * _RB
        for r in range(_RB):
            pltpu.async_copy(x_hbm.at[row0 + r], x_refs[s][r], sem_x[s])

    def wait_x(s):
        for r in range(_RB):
            pltpu.make_async_copy(x_hbm.at[base], x_refs[s][r], sem_x[s]).wait()

    def start_out(bi, s):
        row0 = base + bi * _RB
        for r in range(_RB):
            pltpu.async_copy(acc_refs[s][r], out_hbm.at[row0 + r], sem_o[s])

    def wait_out(s):
        for r in range(_RB):
            pltpu.make_async_copy(acc_refs[s][r], out_hbm.at[base], sem_o[s]).wait()

    start_x(0, 0)

    def pair_body(p, carry):
        for par in (0, 1):
            bi = p * 2 + par
            nbi = jnp.minimum(bi + 1, _NBLK - 1)
            start_x(nbi, 1 - par)

            @pl.when(bi >= 2)
            def _():
                wait_out(par)

            @plsc.parallel_loop(0, _D_F // _L, unroll=4)
            def zero_body(k):
                z = jnp.zeros((_L,), jnp.float32)
                for r in range(_RB):
                    acc_refs[par][r][pl.ds(k * _L, _L)] = z

            wait_x(par)

            @plsc.parallel_loop(0, _GROUPS, unroll=4)
            def scat_body(j):
                o = j * _L
                idx = idx_v[pl.ds(o, _L)]
                sgn = sgn_v[pl.ds(o, _L)]
                for r in range(_RB):
                    v = x_refs[par][r][pl.ds(o, _L)] * sgn
                    plsc.addupdate_scatter(acc_refs[par][r], [idx], v)
            start_out(bi, par)
        return carry

    lax.fori_loop(0, _NBLK // 2, pair_body, 0)
    wait_x(0)
    wait_out(0)
    wait_out(1)


_BM = 512
_BK = 512
_NKB = _D_IN // _BK
_ROW0_BLK = _B_SC // _BM


def _tc_body(i2d_ref, s2d_ref, x_ref, o_ref):
    k = pl.program_id(1)

    @pl.when(k == 0)
    def _():
        o_ref[...] = jnp.zeros_like(o_ref)

    ih = i2d_ref[pl.ds(k * _BK, _BK), :]
    sg = s2d_ref[pl.ds(k * _BK, _BK), :]
    iota = lax.broadcasted_iota(jnp.int32, (_BK, _D_F), 1)
    s_blk = jnp.where(iota == ih, sg, 0.0)
    o_ref[...] += jnp.dot(x_ref[...].astype(jnp.bfloat16),
                          s_blk.astype(jnp.bfloat16),
                          preferred_element_type=jnp.float32)


def _tc_sketch(x, s_hash, i_hash):
    return pl.pallas_call(
        _tc_body,
        grid=(_B_TC // _BM, _NKB),
        in_specs=[
            pl.BlockSpec((_D_IN, 1), lambda i, k: (0, 0)),
            pl.BlockSpec((_D_IN, 1), lambda i, k: (0, 0)),
            pl.BlockSpec((_BM, _BK), lambda i, k: (i + _ROW0_BLK, k)),
        ],
        out_specs=pl.BlockSpec((_BM, _D_F), lambda i, k: (i, 0)),
        out_shape=jax.ShapeDtypeStruct((_B_TC, _D_F), jnp.float32),
    )(i_hash.reshape(_D_IN, 1), s_hash.reshape(_D_IN, 1), x)


def kernel(x, s_hash, i_hash):
    i32 = i_hash.astype(jnp.int32)
    out_sc = _sc_sketch(x, s_hash, i32)
    out_tc = _tc_sketch(x, s_hash, i32)
    return jnp.concatenate([out_sc, out_tc], axis=0)
